# baseline (speedup 1.0000x reference)
import functools

import jax
import jax.numpy as jnp
from jax import lax
from jax.experimental import pallas as pl
from jax.experimental.pallas import tpu as pltpu
from jax.experimental.pallas import tpu_sc as plsc

_pallas_call = pl.pallas_call

_AE16 = [('ae_msg', 30), ('ae_act', 13), ('ae_yesno', 3), ('ae_phase', 4),
         ('ae_cancel', 3)]
_AE8 = [('ae_finish', 3), ('ae_pos', 9), ('ae_opt', 6), ('ae_num', 13),
        ('ae_place', 31), ('ae_attrib', 10)]
_FEAT = [('ce_owner', 2), ('ce_pos', 9), ('ce_over', 2), ('ce_attr', 8),
         ('ce_race', 27), ('ce_level', 14), ('ce_counter', 16), ('ce_neg', 3)]

_EPS = 1e-6


def _lnorm(x, s, b):
    m = jnp.mean(x, -1, keepdims=True)
    d = x - m
    v = jnp.mean(d * d, -1, keepdims=True)
    return d * lax.rsqrt(v + _EPS) * s + b


def _bin_consts():
    p1 = jnp.linspace(0, 8000, 25, dtype=jnp.float32)[1:]
    p2 = jnp.linspace(8000, 32000, 9, dtype=jnp.float32)[1:]
    points = jnp.concatenate([p1, p2])
    intervals = jnp.concatenate([points[0:1], points[1:] - points[:-1]])
    return points.reshape(1, 32), intervals.reshape(1, 32)


def _offsets(tabs):
    offs, t = [], 0
    for _, r in tabs:
        offs.append(t)
        t += r
    return offs, t


def _blockdiag(params, tabs, w, rows_pad):
    offs, total = _offsets(tabs)
    m = jnp.zeros((rows_pad, w * len(tabs)), jnp.float32)
    for j, (n, r) in enumerate(tabs):
        m = m.at[offs[j]:offs[j] + r, j * w:(j + 1) * w].set(params[n])
    return m


def _clip_off(idx, tabs, offs):
    his = jnp.array([r - 1 for _, r in tabs], jnp.int32)
    return jnp.clip(idx, 0, his) + jnp.array(offs, jnp.int32)


def _tables_body(emb, w1, w2, s, b, hw1, hw2, tcard, thist):
    e = emb[...]
    a = jnp.maximum(jnp.dot(e, w1[...], preferred_element_type=jnp.float32), 0.)
    tcard[...] = _lnorm(jnp.dot(a, w2[...], preferred_element_type=jnp.float32),
                        s[...], b[...])
    ah = jnp.maximum(jnp.dot(e, hw1[...], preferred_element_type=jnp.float32), 0.)
    thist[...] = jnp.dot(ah, hw2[...], preferred_element_type=jnp.float32)


def _tables_call(emb, w1, w2, s, b, hw1, hw2):
    R = 128
    n = emb.shape[0] // R
    full = lambda shp: pl.BlockSpec(shp, lambda i: (0, 0))
    return _pallas_call(
        _tables_body,
        grid=(n,),
        in_specs=[pl.BlockSpec((R, 1024), lambda i: (i, 0)),
                  full((1024, 128)), full((128, 32)), full((1, 32)),
                  full((1, 32)), full((1024, 128)), full((128, 128))],
        out_specs=[pl.BlockSpec((R, 32), lambda i: (i, 0)),
                   pl.BlockSpec((R, 128), lambda i: (i, 0))],
        out_shape=[jax.ShapeDtypeStruct((emb.shape[0], 32), jnp.float32),
                   jax.ShapeDtypeStruct((emb.shape[0], 128), jnp.float32)],
    )(emb, w1, w2, s, b, hw1, hw2)


def _c1_body(idg, loci, seqi, fidx, x2, pts, ivs, numw, atkw, defw, typw,
             fs, fb, lstab, fbd, out):
    BB = idg.shape[0]
    T = BB * 80
    x2r = x2[...].reshape(T, 32)
    p = pts[...]
    iv = ivs[...]
    nw = numw[...]

    def num(v):
        bins = jnp.clip((v - p + iv) / iv, 0.0, 1.0)
        return jnp.maximum(jnp.dot(bins, nw, preferred_element_type=jnp.float32), 0.)

    va = x2r[:, 0:1] * 256.0 + x2r[:, 1:2]
    vd = x2r[:, 2:3] * 256.0 + x2r[:, 3:4]
    atk = jnp.dot(num(va), atkw[...], preferred_element_type=jnp.float32)
    dfn = jnp.dot(num(vd), defw[...], preferred_element_type=jnp.float32)
    typ = jnp.dot(x2r[:, 4:32], typw[...], preferred_element_type=jnp.float32)

    li = loci[...].reshape(T, 1)
    si = seqi[...].reshape(T, 1)
    io96 = lax.broadcasted_iota(jnp.int32, (T, 96), 1)
    ohls = ((io96 == li) | (io96 == si + 16)).astype(jnp.float32)
    lsadd = jnp.dot(ohls, lstab[...], preferred_element_type=jnp.float32)

    fr = fidx[...].reshape(T, 8)
    io88 = lax.broadcasted_iota(jnp.int32, (T, 88), 1)
    mh = (io88 == fr[:, 0:1]).astype(jnp.float32)
    for j in range(1, 8):
        mh = mh + (io88 == fr[:, j:j + 1]).astype(jnp.float32)
    feats = jnp.dot(mh, fbd[...], preferred_element_type=jnp.float32)

    xf = jnp.concatenate([feats, atk, dfn, typ], axis=-1)
    xf = _lnorm(xf, fs[...], fb[...])
    tok = jnp.concatenate([idg[...].reshape(T, 32), xf], axis=-1)
    out[...] = (tok + lsadd).reshape(BB, 80, 128)


def _c1_call(idg, loci, seqi, fidx, x2, pts, ivs, numw, atkw, defw, typw,
             fs, fb, lstab, fbd):
    B = idg.shape[0]
    BB = 16
    full = lambda shp: pl.BlockSpec(shp, lambda i: tuple(0 for _ in shp))
    blk3 = lambda d: pl.BlockSpec((BB, 80, d), lambda i: (i, 0, 0))
    return _pallas_call(
        _c1_body,
        grid=(B // BB,),
        in_specs=[blk3(32), blk3(1), blk3(1), blk3(8), blk3(32),
                  full((1, 32)), full((1, 32)), full((32, 16)), full((16, 8)),
                  full((16, 8)), full((28, 16)), full((1, 96)), full((1, 96)),
                  full((96, 128)), full((88, 64))],
        out_specs=blk3(128),
        out_shape=jax.ShapeDtypeStruct((B, 80, 128), jnp.float32),
    )(idg, loci, seqi, fidx, x2, pts, ivs, numw, atkw, defw, typw, fs, fb,
      lstab, fbd)


def _attn(h, wq, bq, wk, bk, wv, bv, wo, bo, BB, T):
    q = (jnp.dot(h, wq, preferred_element_type=jnp.float32) + bq).astype(
        jnp.bfloat16)
    k = (jnp.dot(h, wk, preferred_element_type=jnp.float32) + bk).astype(
        jnp.bfloat16)
    v = (jnp.dot(h, wv, preferred_element_type=jnp.float32) + bv).astype(
        jnp.bfloat16)
    madd = jnp.where(lax.broadcasted_iota(jnp.int32, (1, T), 1) >= 81,
                     -1e30, 0.0)
    outs = []
    for bi in range(BB):
        heads = []
        for hd in range(2):
            qb = q[bi * T:(bi + 1) * T, hd * 64:(hd + 1) * 64]
            kb = k[bi * T:(bi + 1) * T, hd * 64:(hd + 1) * 64]
            vb = v[bi * T:(bi + 1) * T, hd * 64:(hd + 1) * 64]
            a = lax.dot_general(qb, kb, (((1,), (1,)), ((), ())),
                                preferred_element_type=jnp.float32) * 0.125
            e = jnp.exp(a + madd)
            a = (e * (1.0 / jnp.sum(e, -1, keepdims=True))).astype(jnp.bfloat16)
            heads.append(jnp.dot(a, vb, preferred_element_type=jnp.float32))
        outs.append(jnp.concatenate(heads, axis=-1))
    o = jnp.concatenate(outs, axis=0).astype(jnp.bfloat16)
    return jnp.dot(o, wo, preferred_element_type=jnp.float32) + bo


def _c2_body(x0, *refs):
    BB, T, C = x0.shape
    out = refs[-1]
    w = [r[...] for r in refs[:-1]]
    x = x0[...].reshape(BB * T, C)
    for i in range(2):
        (wq, bq, wk, bk, wv, bv, wo, bo, l1s, l1b, l2s, l2b, m1, m2) = \
            w[i * 14:(i + 1) * 14]
        h = _lnorm(x, l1s, l1b).astype(jnp.bfloat16)
        x = x + _attn(h, wq, bq, wk, bk, wv, bv, wo, bo, BB, T)
        h = _lnorm(x, l2s, l2b).astype(jnp.bfloat16)
        h = jnp.maximum(jnp.dot(h, m1, preferred_element_type=jnp.float32),
                        0.).astype(jnp.bfloat16)
        x = x + jnp.dot(h, m2, preferred_element_type=jnp.float32)
    es, eb = w[28], w[29]
    out[...] = _lnorm(x, es, eb).reshape(BB, T, C)


def _c2_call(x0, wlist):
    B, T, C = x0.shape
    BB = 16
    full = lambda shp: pl.BlockSpec(shp, lambda i: tuple(0 for _ in shp))
    blk = pl.BlockSpec((BB, T, C), lambda i: (i, 0, 0))
    return _pallas_call(
        _c2_body,
        grid=(B // BB,),
        in_specs=[blk] + [full(w.shape) for w in wlist],
        out_specs=blk,
        out_shape=jax.ShapeDtypeStruct((B, T, C), jnp.float32),
    )(x0, *wlist)


def _h_body(histg, a16, a8, bd16, bd8, out):
    BB, NH = a16.shape[0], a16.shape[1]
    T = BB * NH
    i16 = a16[...].reshape(T, 5)
    i8 = a8[...].reshape(T, 6)
    io56 = lax.broadcasted_iota(jnp.int32, (T, 56), 1)
    mh16 = (io56 == i16[:, 0:1]).astype(jnp.float32)
    for j in range(1, 5):
        mh16 = mh16 + (io56 == i16[:, j:j + 1]).astype(jnp.float32)
    o16 = jnp.dot(mh16, bd16[...], preferred_element_type=jnp.float32)
    io80 = lax.broadcasted_iota(jnp.int32, (T, 80), 1)
    mh8 = (io80 == i8[:, 0:1]).astype(jnp.float32)
    for j in range(1, 6):
        mh8 = mh8 + (io80 == i8[:, j:j + 1]).astype(jnp.float32)
    o8 = jnp.dot(mh8, bd8[...], preferred_element_type=jnp.float32)
    r = jnp.concatenate([histg[...].reshape(T, 128), o16, o8], axis=-1)
    out[...] = r.reshape(BB, NH, 256)


def _h_call(histg, a16, a8, bd16, bd8):
    B, NH = histg.shape[0], histg.shape[1]
    BB = 32
    full = lambda shp: pl.BlockSpec(shp, lambda i: tuple(0 for _ in shp))
    blk3 = lambda d: pl.BlockSpec((BB, NH, d), lambda i: (i, 0, 0))
    return _pallas_call(
        _h_body,
        grid=(B // BB,),
        in_specs=[blk3(128), blk3(5), blk3(6), full((56, 80)), full((80, 48))],
        out_specs=blk3(256),
        out_shape=jax.ShapeDtypeStruct((B, NH, 256), jnp.float32),
    )(histg, a16, a8, bd16, bd8)


def _d_body(g1, g2o, g3c, hnd, pts, ivs, numw, lpw, olpw, ge4bd, cntw, hndw,
            gs, gb, w1, w2, fc, os_, ob_, out):
    B = g1.shape[0]
    p = pts[...]
    iv = ivs[...]
    nw = numw[...]
    g = g1[...]

    def num(v):
        bins = jnp.clip((v - p + iv) / iv, 0.0, 1.0)
        return jnp.maximum(jnp.dot(bins, nw, preferred_element_type=jnp.float32), 0.)

    lp = jnp.dot(num(g[:, 0:1] * 256.0 + g[:, 1:2]), lpw[...],
                 preferred_element_type=jnp.float32)
    olp = jnp.dot(num(g[:, 2:3] * 256.0 + g[:, 3:4]), olpw[...],
                  preferred_element_type=jnp.float32)

    g2 = g2o[...]
    io40 = lax.broadcasted_iota(jnp.int32, (B, 40), 1)
    mh4 = (io40 == g2[:, 0:1]).astype(jnp.float32)
    for j in range(1, 4):
        mh4 = mh4 + (io40 == g2[:, j:j + 1]).astype(jnp.float32)
    ge4 = jnp.dot(mh4, ge4bd[...], preferred_element_type=jnp.float32)

    io104 = lax.broadcasted_iota(jnp.int32, (B, 104), 1)
    cw = cntw[...]
    hw = hndw[...]
    g3 = g3c[...]
    pieces = [lp, olp, ge4]
    for j in range(14):
        oh = (io104 == g3[:, j:j + 1]).astype(jnp.float32)
        pieces.append(jnp.dot(oh, cw, preferred_element_type=jnp.float32))
    hn = hnd[...]
    for j in range(2):
        oh = (io104 == hn[:, j:j + 1]).astype(jnp.float32)
        pieces.append(jnp.dot(oh, hw, preferred_element_type=jnp.float32))
    xg = jnp.concatenate(pieces, axis=-1)
    xg = _lnorm(xg, gs[...], gb[...])
    h = jnp.maximum(jnp.dot(xg, w1[...], preferred_element_type=jnp.float32), 0.)
    fg = xg + jnp.dot(h, w2[...], preferred_element_type=jnp.float32)
    fg = jnp.dot(fg, fc[...], preferred_element_type=jnp.float32)
    out[...] = _lnorm(fg, os_[...], ob_[...])


def _d_call(g1, g2o, g3c, hnd, pts, ivs, numw, lpw, olpw, ge4bd, cntw, hndw,
            gs, gb, w1, w2, fc, os_, ob_):
    B = g1.shape[0]
    full = lambda shp: pl.BlockSpec(shp, lambda: tuple(0 for _ in shp))
    args = (g1, g2o, g3c, hnd, pts, ivs, numw, lpw, olpw, ge4bd, cntw, hndw,
            gs, gb, w1, w2, fc, os_, ob_)
    return _pallas_call(
        _d_body,
        in_specs=[full(a.shape) for a in args],
        out_specs=full((B, 128)),
        out_shape=jax.ShapeDtypeStruct((B, 128), jnp.float32),
    )(*args)


def _pick_chunk(n, d):
    limit = (131072 // (d * 4)) // 8 * 8
    c = min(n, limit)
    c -= c % 8
    while c > 8 and n % c:
        c -= 8
    return c


def _sc_gather2(table_a, idx_a, table_b, idx_b):
    V = 1024
    Da = table_a.shape[1]
    Na = idx_a.shape[0]
    Nb = idx_b.shape[0]
    NS = 16
    npa = Na // NS
    npb = Nb // NS
    CA = 640
    CB = 256
    HW = 64
    tab_a64 = jnp.pad(table_a, ((0, 0), (0, HW - Da)))
    mesh = plsc.VectorSubcoreMesh(core_axis_name="c", subcore_axis_name="s")

    @functools.partial(
        pl.kernel, mesh=mesh,
        out_type=[jax.ShapeDtypeStruct((Na, Da), jnp.float32),
                  jax.ShapeDtypeStruct((Nb, 2 * HW), jnp.float32)],
        compiler_params=pltpu.CompilerParams(use_tc_tiling_on_sc=False,
                                             needs_layout_passes=False),
        scratch_types=[pltpu.VMEM((V, HW), jnp.float32),
                       pltpu.VMEM((npa,), jnp.int32),
                       pltpu.VMEM((CA, Da), jnp.float32),
                       pltpu.VMEM((CB, HW), jnp.float32)],
    )
    def k(ia_hbm, ta_hbm, ib_hbm, tb_hbm, oa_hbm, ob_hbm,
          tab_v, idx_v, oa_v, ob_v):
        cid = lax.axis_index("c")
        sid = lax.axis_index("s")
        iot = lax.broadcasted_iota(jnp.int32, (16,), 0)

        @pl.when(cid == 0)
        def _card():
            pltpu.sync_copy(ta_hbm, tab_v)
            pltpu.sync_copy(ia_hbm.at[pl.ds(sid * npa, npa)], idx_v)
            for ch in range(npa // CA):
                def body(kk, _):
                    ids = idx_v[pl.ds(ch * CA + kk * 16, 16)]
                    rows = iot + kk * 16
                    for c in range(Da):
                        cv = jnp.full((16,), c, jnp.int32)
                        vals = plsc.load_gather(tab_v, [ids, cv])
                        plsc.store_scatter(oa_v, [rows, cv], vals)
                    return 0
                lax.fori_loop(0, CA // 16, body, 0)
                pltpu.sync_copy(oa_v, oa_hbm.at[pl.ds(sid * npa + ch * CA, CA)])

        @pl.when(cid == 1)
        def _hist():
            pltpu.sync_copy(ib_hbm.at[pl.ds(sid * npb, npb)],
                            idx_v.at[pl.ds(0, npb)])
            for half in range(2):
                pltpu.sync_copy(tb_hbm.at[:, pl.ds(half * HW, HW)], tab_v)
                for ch in range(npb // CB):
                    def body(kk, _):
                        ids = idx_v[pl.ds(ch * CB + kk * 16, 16)]
                        rows = iot + kk * 16
                        for c in range(HW):
                            cv = jnp.full((16,), c, jnp.int32)
                            vals = plsc.load_gather(tab_v, [ids, cv])
                            plsc.store_scatter(ob_v, [rows, cv], vals)
                        return 0
                    lax.fori_loop(0, CB // 16, body, 0)
                    pltpu.sync_copy(
                        ob_v, ob_hbm.at[pl.ds(sid * npb + ch * CB, CB),
                                        pl.ds(half * HW, HW)])

    return k(idx_a, tab_a64, idx_b, table_b)


def kernel(cards_, global_, actions_, h_actions_, params):
    p = params
    B, NCARD = cards_.shape[0], cards_.shape[1]
    NH = h_actions_.shape[1]
    pts, ivs = _bin_consts()

    emb = jnp.pad(p['id_embed'], ((0, 23), (0, 0)))
    tcard, thist = _tables_call(
        emb, p['ce_id_w1'], p['ce_id_w2'],
        p['ce_id_ln_s'].reshape(1, 32), p['ce_id_ln_b'].reshape(1, 32),
        p['h_id_w1'], p['h_id_w2'])

    ids_card = jnp.clip(cards_[:, :, 0] * 256 + cards_[:, :, 1], 0, 1000)
    x1 = cards_[:, :, 2:12]
    h = h_actions_.astype(jnp.int32)
    ids_h = jnp.clip(h[:, :, 0] * 256 + h[:, :, 1], 0, 1000)
    ae = h[:, :, 2:13]
    g2 = global_[:, 4:8].astype(jnp.int32)
    g3 = jnp.clip(global_[:, 8:22].astype(jnp.int32), 0, 99)

    g32, ghist = _sc_gather2(tcard, ids_card.reshape(-1),
                             thist, ids_h.reshape(-1))
    id32g = g32.reshape(B, NCARD, 32)
    histg = ghist.reshape(B, NH, 128)

    loc_i = jnp.clip(x1[:, :, 0:1], 0, 8)
    seq_i = jnp.clip(x1[:, :, 1:2], 0, 75)
    ft_offs, _ = _offsets(_FEAT)
    fidx = _clip_off(x1[:, :, 2:10], _FEAT, ft_offs)

    locp = jnp.pad(_lnorm(p['ce_loc_embed'], p['ce_loc_ln_s'],
                          p['ce_loc_ln_b']), ((0, 7), (0, 0)))
    seqp = jnp.pad(_lnorm(p['ce_seq_embed'], p['ce_seq_ln_s'],
                          p['ce_seq_ln_b']), ((0, 4), (0, 0)))
    lstab = jnp.concatenate([locp, seqp], 0)
    fbd = _blockdiag(p, _FEAT, 8, 88)

    x2 = cards_[:, :, 12:41].astype(jnp.float32)
    x2p = jnp.pad(x2, ((0, 0), (0, 0), (0, 3)))
    typw = jnp.pad(p['ce_type_w'], ((0, 3), (0, 0)))
    tokens = _c1_call(id32g, loc_i, seq_i, fidx, x2p, pts, ivs,
                      p['ce_num_w'], p['ce_atk_w'], p['ce_def_w'], typw,
                      p['ce_f_ln_s'].reshape(1, 96),
                      p['ce_f_ln_b'].reshape(1, 96), lstab, fbd)

    x0 = jnp.concatenate(
        [jnp.broadcast_to(p['g_card_embed'][None], (B, 1, 128)), tokens,
         jnp.zeros((B, 7, 128), jnp.float32)], axis=1)
    wlist = []
    for i in range(2):
        for nme in ['wq', 'bq', 'wk', 'bk', 'wv', 'bv', 'wo', 'bo']:
            w = p[f'tl{i}_{nme}']
            wlist.append(w.reshape(1, 128) if w.ndim == 1
                         else w.astype(jnp.bfloat16))
        for nme in ['ln1_s', 'ln1_b', 'ln2_s', 'ln2_b']:
            wlist.append(p[f'tl{i}_{nme}'].reshape(1, 128))
        wlist.append(p[f'tl{i}_mw1'].astype(jnp.bfloat16))
        wlist.append(p[f'tl{i}_mw2'].astype(jnp.bfloat16))
    wlist.append(p['enc_ln_s'].reshape(1, 128))
    wlist.append(p['enc_ln_b'].reshape(1, 128))
    f_cards = _c2_call(x0, wlist)[:, :81]

    ae16_offs, _ = _offsets(_AE16)
    ae8_offs, _ = _offsets(_AE8)
    a16 = _clip_off(ae[:, :, 0:5], _AE16, ae16_offs)
    a8 = _clip_off(ae[:, :, 5:11], _AE8, ae8_offs)
    bd16 = _blockdiag(p, _AE16, 16, 56)
    bd8 = _blockdiag(p, _AE8, 8, 80)
    f_h = _h_call(histg, a16, a8, bd16, bd8)

    ge4tabs = [('ge_turn', 20), ('ge_phase', 11), ('ge_first', 2),
               ('ge_myturn', 2)]
    ge4_offs, _ = _offsets(ge4tabs)
    g2o = _clip_off(g2, ge4tabs, ge4_offs)
    ge4bd = _blockdiag(p, ge4tabs, 16, 40)
    hnd = jnp.concatenate([g3[:, 1:2], g3[:, 8:9]], axis=1)
    cntw = jnp.pad(p['ge_count'], ((0, 4), (0, 0)))
    hndw = jnp.pad(p['ge_hand'], ((0, 4), (0, 0)))
    g1f = global_[:, :4].astype(jnp.float32)
    fg = _d_call(g1f, g2o, g3, hnd, pts, ivs, p['ge_num_w'], p['ge_lp_w'],
                 p['ge_olp_w'], ge4bd, cntw, hndw,
                 p['ge_ln_s'].reshape(1, 256), p['ge_ln_b'].reshape(1, 256),
                 p['g_mlp_w1'], p['g_mlp_w2'], p['g_fc_w'],
                 p['g_ln_s'].reshape(1, 128), p['g_ln_b'].reshape(1, 128))

    return (f_cards, fg, f_h)


# speedup vs baseline: 1.2899x; 1.2899x over previous
import functools

import jax
import jax.numpy as jnp
from jax import lax
from jax.experimental import pallas as pl
from jax.experimental.pallas import tpu as pltpu
from jax.experimental.pallas import tpu_sc as plsc

_pallas_call = pl.pallas_call

_AE16 = [('ae_msg', 30), ('ae_act', 13), ('ae_yesno', 3), ('ae_phase', 4),
         ('ae_cancel', 3)]
_AE8 = [('ae_finish', 3), ('ae_pos', 9), ('ae_opt', 6), ('ae_num', 13),
        ('ae_place', 31), ('ae_attrib', 10)]
_FEAT = [('ce_owner', 2), ('ce_pos', 9), ('ce_over', 2), ('ce_attr', 8),
         ('ce_race', 27), ('ce_level', 14), ('ce_counter', 16), ('ce_neg', 3)]

_EPS = 1e-6


def _lnorm(x, s, b):
    m = jnp.mean(x, -1, keepdims=True)
    d = x - m
    v = jnp.mean(d * d, -1, keepdims=True)
    return d * lax.rsqrt(v + _EPS) * s + b


def _bin_consts():
    p1 = jnp.linspace(0, 8000, 25, dtype=jnp.float32)[1:]
    p2 = jnp.linspace(8000, 32000, 9, dtype=jnp.float32)[1:]
    points = jnp.concatenate([p1, p2])
    intervals = jnp.concatenate([points[0:1], points[1:] - points[:-1]])
    return points.reshape(1, 32), intervals.reshape(1, 32)


def _offsets(tabs):
    offs, t = [], 0
    for _, r in tabs:
        offs.append(t)
        t += r
    return offs, t


def _blockdiag(params, tabs, w, rows_pad):
    offs, total = _offsets(tabs)
    m = jnp.zeros((rows_pad, w * len(tabs)), jnp.float32)
    for j, (n, r) in enumerate(tabs):
        m = m.at[offs[j]:offs[j] + r, j * w:(j + 1) * w].set(params[n])
    return m


def _clip_off(idx, tabs, offs):
    his = jnp.array([r - 1 for _, r in tabs], jnp.int32)
    return jnp.clip(idx, 0, his) + jnp.array(offs, jnp.int32)


def _tables_body(emb, w1, w2, s, b, hw1, hw2, tcard, thist):
    e = emb[...]
    a = jnp.maximum(jnp.dot(e, w1[...], preferred_element_type=jnp.float32), 0.)
    tcard[...] = _lnorm(jnp.dot(a, w2[...], preferred_element_type=jnp.float32),
                        s[...], b[...])
    ah = jnp.maximum(jnp.dot(e, hw1[...], preferred_element_type=jnp.float32), 0.)
    thist[...] = jnp.dot(ah, hw2[...], preferred_element_type=jnp.float32)


def _tables_call(emb, w1, w2, s, b, hw1, hw2):
    R = 128
    n = emb.shape[0] // R
    full = lambda shp: pl.BlockSpec(shp, lambda i: (0, 0))
    return _pallas_call(
        _tables_body,
        grid=(n,),
        in_specs=[pl.BlockSpec((R, 1024), lambda i: (i, 0)),
                  full((1024, 128)), full((128, 32)), full((1, 32)),
                  full((1, 32)), full((1024, 128)), full((128, 128))],
        out_specs=[pl.BlockSpec((R, 32), lambda i: (i, 0)),
                   pl.BlockSpec((R, 128), lambda i: (i, 0))],
        out_shape=[jax.ShapeDtypeStruct((emb.shape[0], 32), jnp.float32),
                   jax.ShapeDtypeStruct((emb.shape[0], 128), jnp.float32)],
    )(emb, w1, w2, s, b, hw1, hw2)


def _c1_body(idg, loci, seqi, fidx, x2, pts, ivs, numw, atkw, defw, typw,
             fs, fb, lstab, fbd, out):
    BB = idg.shape[0]
    T = BB * 80
    x2r = x2[...].reshape(T, 32)
    p = pts[...]
    iv = ivs[...]
    nw = numw[...]

    def num(v):
        bins = jnp.clip((v - p + iv) / iv, 0.0, 1.0)
        return jnp.maximum(jnp.dot(bins, nw, preferred_element_type=jnp.float32), 0.)

    va = x2r[:, 0:1] * 256.0 + x2r[:, 1:2]
    vd = x2r[:, 2:3] * 256.0 + x2r[:, 3:4]
    atk = jnp.dot(num(va), atkw[...], preferred_element_type=jnp.float32)
    dfn = jnp.dot(num(vd), defw[...], preferred_element_type=jnp.float32)
    typ = jnp.dot(x2r[:, 4:32], typw[...], preferred_element_type=jnp.float32)

    li = loci[...].reshape(T, 1)
    si = seqi[...].reshape(T, 1)
    io96 = lax.broadcasted_iota(jnp.int32, (T, 96), 1)
    ohls = ((io96 == li) | (io96 == si + 16)).astype(jnp.float32)
    lsadd = jnp.dot(ohls, lstab[...], preferred_element_type=jnp.float32)

    fr = fidx[...].reshape(T, 8)
    io88 = lax.broadcasted_iota(jnp.int32, (T, 88), 1)
    mh = (io88 == fr[:, 0:1]).astype(jnp.float32)
    for j in range(1, 8):
        mh = mh + (io88 == fr[:, j:j + 1]).astype(jnp.float32)
    feats = jnp.dot(mh, fbd[...], preferred_element_type=jnp.float32)

    xf = jnp.concatenate([feats, atk, dfn, typ], axis=-1)
    xf = _lnorm(xf, fs[...], fb[...])
    tok = jnp.concatenate([idg[...].reshape(T, 32), xf], axis=-1)
    out[...] = (tok + lsadd).reshape(BB, 80, 128)


def _c1_call(idg, loci, seqi, fidx, x2, pts, ivs, numw, atkw, defw, typw,
             fs, fb, lstab, fbd):
    B = idg.shape[0]
    BB = 16
    full = lambda shp: pl.BlockSpec(shp, lambda i: tuple(0 for _ in shp))
    blk3 = lambda d: pl.BlockSpec((BB, 80, d), lambda i: (i, 0, 0))
    return _pallas_call(
        _c1_body,
        grid=(B // BB,),
        in_specs=[blk3(32), blk3(1), blk3(1), blk3(8), blk3(32),
                  full((1, 32)), full((1, 32)), full((32, 16)), full((16, 8)),
                  full((16, 8)), full((28, 16)), full((1, 96)), full((1, 96)),
                  full((96, 128)), full((88, 64))],
        out_specs=blk3(128),
        out_shape=jax.ShapeDtypeStruct((B, 80, 128), jnp.float32),
    )(idg, loci, seqi, fidx, x2, pts, ivs, numw, atkw, defw, typw, fs, fb,
      lstab, fbd)


def _attn(h, wq, bq, wk, bk, wv, bv, wo, bo, BB, T):
    q = jnp.dot(h, wq, preferred_element_type=jnp.float32) + bq
    k = jnp.dot(h, wk, preferred_element_type=jnp.float32) + bk
    v = jnp.dot(h, wv, preferred_element_type=jnp.float32) + bv
    madd = jnp.where(lax.broadcasted_iota(jnp.int32, (1, T), 1) >= 81,
                     -1e30, 0.0)
    outs = []
    for bi in range(BB):
        heads = []
        for hd in range(2):
            qb = q[bi * T:(bi + 1) * T, hd * 64:(hd + 1) * 64]
            kb = k[bi * T:(bi + 1) * T, hd * 64:(hd + 1) * 64]
            vb = v[bi * T:(bi + 1) * T, hd * 64:(hd + 1) * 64]
            a = lax.dot_general(qb, kb, (((1,), (1,)), ((), ())),
                                preferred_element_type=jnp.float32) * 0.125
            e = jnp.exp(a + madd)
            a = e * (1.0 / jnp.sum(e, -1, keepdims=True))
            heads.append(jnp.dot(a, vb, preferred_element_type=jnp.float32))
        outs.append(jnp.concatenate(heads, axis=-1))
    o = jnp.concatenate(outs, axis=0)
    return jnp.dot(o, wo, preferred_element_type=jnp.float32) + bo


def _c2_body(x0, *refs):
    BB, T, C = x0.shape
    out = refs[-1]
    w = [r[...] for r in refs[:-1]]
    x = x0[...].reshape(BB * T, C)
    for i in range(2):
        (wq, bq, wk, bk, wv, bv, wo, bo, l1s, l1b, l2s, l2b, m1, m2) = \
            w[i * 14:(i + 1) * 14]
        h = _lnorm(x, l1s, l1b)
        x = x + _attn(h, wq, bq, wk, bk, wv, bv, wo, bo, BB, T)
        h = _lnorm(x, l2s, l2b)
        h = jnp.maximum(jnp.dot(h, m1, preferred_element_type=jnp.float32), 0.)
        x = x + jnp.dot(h, m2, preferred_element_type=jnp.float32)
    es, eb = w[28], w[29]
    out[...] = _lnorm(x, es, eb).reshape(BB, T, C)[:, :81, :]


def _c2_call(x0, wlist):
    B, T, C = x0.shape
    BB = 16
    full = lambda shp: pl.BlockSpec(shp, lambda i: tuple(0 for _ in shp))
    blk = pl.BlockSpec((BB, T, C), lambda i: (i, 0, 0))
    return _pallas_call(
        _c2_body,
        grid=(B // BB,),
        in_specs=[blk] + [full(w.shape) for w in wlist],
        out_specs=pl.BlockSpec((BB, 81, C), lambda i: (i, 0, 0)),
        out_shape=jax.ShapeDtypeStruct((B, 81, C), jnp.float32),
    )(x0, *wlist)


def _h_body(histg, a16, a8, bd16, bd8, out):
    BB, NH = a16.shape[0], a16.shape[1]
    T = BB * NH
    i16 = a16[...].reshape(T, 5)
    i8 = a8[...].reshape(T, 6)
    io56 = lax.broadcasted_iota(jnp.int32, (T, 56), 1)
    mh16 = (io56 == i16[:, 0:1]).astype(jnp.float32)
    for j in range(1, 5):
        mh16 = mh16 + (io56 == i16[:, j:j + 1]).astype(jnp.float32)
    o16 = jnp.dot(mh16, bd16[...], preferred_element_type=jnp.float32)
    io80 = lax.broadcasted_iota(jnp.int32, (T, 80), 1)
    mh8 = (io80 == i8[:, 0:1]).astype(jnp.float32)
    for j in range(1, 6):
        mh8 = mh8 + (io80 == i8[:, j:j + 1]).astype(jnp.float32)
    o8 = jnp.dot(mh8, bd8[...], preferred_element_type=jnp.float32)
    r = jnp.concatenate([histg[...].reshape(T, 128), o16, o8], axis=-1)
    out[...] = r.reshape(BB, NH, 256)


def _h_call(histg, a16, a8, bd16, bd8):
    B, NH = histg.shape[0], histg.shape[1]
    BB = 32
    full = lambda shp: pl.BlockSpec(shp, lambda i: tuple(0 for _ in shp))
    blk3 = lambda d: pl.BlockSpec((BB, NH, d), lambda i: (i, 0, 0))
    return _pallas_call(
        _h_body,
        grid=(B // BB,),
        in_specs=[blk3(128), blk3(5), blk3(6), full((56, 80)), full((80, 48))],
        out_specs=blk3(256),
        out_shape=jax.ShapeDtypeStruct((B, NH, 256), jnp.float32),
    )(histg, a16, a8, bd16, bd8)


def _d_body(g1, g2o, g3c, hnd, pts, ivs, numw, lpw, olpw, ge4bd, cntw, hndw,
            gs, gb, w1, w2, fc, os_, ob_, out):
    B = g1.shape[0]
    p = pts[...]
    iv = ivs[...]
    nw = numw[...]
    g = g1[...]

    def num(v):
        bins = jnp.clip((v - p + iv) / iv, 0.0, 1.0)
        return jnp.maximum(jnp.dot(bins, nw, preferred_element_type=jnp.float32), 0.)

    lp = jnp.dot(num(g[:, 0:1] * 256.0 + g[:, 1:2]), lpw[...],
                 preferred_element_type=jnp.float32)
    olp = jnp.dot(num(g[:, 2:3] * 256.0 + g[:, 3:4]), olpw[...],
                  preferred_element_type=jnp.float32)

    g2 = g2o[...]
    io40 = lax.broadcasted_iota(jnp.int32, (B, 40), 1)
    mh4 = (io40 == g2[:, 0:1]).astype(jnp.float32)
    for j in range(1, 4):
        mh4 = mh4 + (io40 == g2[:, j:j + 1]).astype(jnp.float32)
    ge4 = jnp.dot(mh4, ge4bd[...], preferred_element_type=jnp.float32)

    io104 = lax.broadcasted_iota(jnp.int32, (B, 104), 1)
    cw = cntw[...]
    hw = hndw[...]
    g3 = g3c[...]
    pieces = [lp, olp, ge4]
    for j in range(14):
        oh = (io104 == g3[:, j:j + 1]).astype(jnp.float32)
        pieces.append(jnp.dot(oh, cw, preferred_element_type=jnp.float32))
    hn = hnd[...]
    for j in range(2):
        oh = (io104 == hn[:, j:j + 1]).astype(jnp.float32)
        pieces.append(jnp.dot(oh, hw, preferred_element_type=jnp.float32))
    xg = jnp.concatenate(pieces, axis=-1)
    xg = _lnorm(xg, gs[...], gb[...])
    h = jnp.maximum(jnp.dot(xg, w1[...], preferred_element_type=jnp.float32), 0.)
    fg = xg + jnp.dot(h, w2[...], preferred_element_type=jnp.float32)
    fg = jnp.dot(fg, fc[...], preferred_element_type=jnp.float32)
    out[...] = _lnorm(fg, os_[...], ob_[...])


def _d_call(g1, g2o, g3c, hnd, pts, ivs, numw, lpw, olpw, ge4bd, cntw, hndw,
            gs, gb, w1, w2, fc, os_, ob_):
    B = g1.shape[0]
    full = lambda shp: pl.BlockSpec(shp, lambda: tuple(0 for _ in shp))
    args = (g1, g2o, g3c, hnd, pts, ivs, numw, lpw, olpw, ge4bd, cntw, hndw,
            gs, gb, w1, w2, fc, os_, ob_)
    return _pallas_call(
        _d_body,
        in_specs=[full(a.shape) for a in args],
        out_specs=full((B, 128)),
        out_shape=jax.ShapeDtypeStruct((B, 128), jnp.float32),
    )(*args)


def _pick_chunk(n, d):
    limit = (131072 // (d * 4)) // 8 * 8
    c = min(n, limit)
    c -= c % 8
    while c > 8 and n % c:
        c -= 8
    return c


def _sc_gather2(table_a, idx_a, table_b, idx_b):
    V = 1024
    Da = table_a.shape[1]
    Na = idx_a.shape[0]
    Nb = idx_b.shape[0]
    NS = 16
    npa = Na // NS
    npb = Nb // NS
    CA = 1280
    CB = 256
    HW = 64
    tab_a64 = jnp.pad(table_a, ((0, 0), (0, HW - Da)))
    mesh = plsc.VectorSubcoreMesh(core_axis_name="c", subcore_axis_name="s")

    @functools.partial(
        pl.kernel, mesh=mesh,
        out_type=[jax.ShapeDtypeStruct((Na, Da), jnp.float32),
                  jax.ShapeDtypeStruct((Nb, 2 * HW), jnp.float32)],
        compiler_params=pltpu.CompilerParams(use_tc_tiling_on_sc=False,
                                             needs_layout_passes=False),
        scratch_types=[pltpu.VMEM((V, HW), jnp.float32),
                       pltpu.VMEM((npa,), jnp.int32),
                       pltpu.VMEM((CA, Da), jnp.float32),
                       pltpu.VMEM((CB, HW), jnp.float32)],
    )
    def k(ia_hbm, ta_hbm, ib_hbm, tb_hbm, oa_hbm, ob_hbm,
          tab_v, idx_v, oa_v, ob_v):
        cid = lax.axis_index("c")
        sid = lax.axis_index("s")
        iot = lax.broadcasted_iota(jnp.int32, (16,), 0)

        @pl.when(cid == 0)
        def _card():
            pltpu.sync_copy(ta_hbm, tab_v)
            pltpu.sync_copy(ia_hbm.at[pl.ds(sid * npa, npa)], idx_v)
            for ch in range(npa // CA):
                def body(kk, _):
                    ids = idx_v[pl.ds(ch * CA + kk * 16, 16)]
                    rows = iot + kk * 16
                    for c in range(Da):
                        cv = jnp.full((16,), c, jnp.int32)
                        vals = plsc.load_gather(tab_v, [ids, cv])
                        plsc.store_scatter(oa_v, [rows, cv], vals)
                    return 0
                lax.fori_loop(0, CA // 16, body, 0)
                pltpu.sync_copy(oa_v, oa_hbm.at[pl.ds(sid * npa + ch * CA, CA)])

        @pl.when(cid == 1)
        def _hist():
            pltpu.sync_copy(ib_hbm.at[pl.ds(sid * npb, npb)],
                            idx_v.at[pl.ds(0, npb)])
            for half in range(2):
                pltpu.sync_copy(tb_hbm.at[:, pl.ds(half * HW, HW)], tab_v)
                for ch in range(npb // CB):
                    def body(kk, _):
                        ids = idx_v[pl.ds(ch * CB + kk * 16, 16)]
                        rows = iot + kk * 16
                        for c in range(HW):
                            cv = jnp.full((16,), c, jnp.int32)
                            vals = plsc.load_gather(tab_v, [ids, cv])
                            plsc.store_scatter(ob_v, [rows, cv], vals)
                        return 0
                    lax.fori_loop(0, CB // 16, body, 0)
                    pltpu.sync_copy(
                        ob_v, ob_hbm.at[pl.ds(sid * npb + ch * CB, CB),
                                        pl.ds(half * HW, HW)])

    return k(idx_a, tab_a64, idx_b, table_b)


def kernel(cards_, global_, actions_, h_actions_, params):
    p = params
    B, NCARD = cards_.shape[0], cards_.shape[1]
    NH = h_actions_.shape[1]
    pts, ivs = _bin_consts()

    emb = jnp.pad(p['id_embed'], ((0, 23), (0, 0)))
    tcard, thist = _tables_call(
        emb, p['ce_id_w1'], p['ce_id_w2'],
        p['ce_id_ln_s'].reshape(1, 32), p['ce_id_ln_b'].reshape(1, 32),
        p['h_id_w1'], p['h_id_w2'])

    ids_card = jnp.clip(cards_[:, :, 0] * 256 + cards_[:, :, 1], 0, 1000)
    x1 = cards_[:, :, 2:12]
    h = h_actions_.astype(jnp.int32)
    ids_h = jnp.clip(h[:, :, 0] * 256 + h[:, :, 1], 0, 1000)
    ae = h[:, :, 2:13]
    g2 = global_[:, 4:8].astype(jnp.int32)
    g3 = jnp.clip(global_[:, 8:22].astype(jnp.int32), 0, 99)

    g32, ghist = _sc_gather2(tcard, ids_card.reshape(-1),
                             thist, ids_h.reshape(-1))
    id32g = g32.reshape(B, NCARD, 32)
    histg = ghist.reshape(B, NH, 128)

    loc_i = jnp.clip(x1[:, :, 0:1], 0, 8)
    seq_i = jnp.clip(x1[:, :, 1:2], 0, 75)
    ft_offs, _ = _offsets(_FEAT)
    fidx = _clip_off(x1[:, :, 2:10], _FEAT, ft_offs)

    locp = jnp.pad(_lnorm(p['ce_loc_embed'], p['ce_loc_ln_s'],
                          p['ce_loc_ln_b']), ((0, 7), (0, 0)))
    seqp = jnp.pad(_lnorm(p['ce_seq_embed'], p['ce_seq_ln_s'],
                          p['ce_seq_ln_b']), ((0, 4), (0, 0)))
    lstab = jnp.concatenate([locp, seqp], 0)
    fbd = _blockdiag(p, _FEAT, 8, 88)

    x2 = cards_[:, :, 12:41].astype(jnp.float32)
    x2p = jnp.pad(x2, ((0, 0), (0, 0), (0, 3)))
    typw = jnp.pad(p['ce_type_w'], ((0, 3), (0, 0)))
    tokens = _c1_call(id32g, loc_i, seq_i, fidx, x2p, pts, ivs,
                      p['ce_num_w'], p['ce_atk_w'], p['ce_def_w'], typw,
                      p['ce_f_ln_s'].reshape(1, 96),
                      p['ce_f_ln_b'].reshape(1, 96), lstab, fbd)

    x0 = jnp.concatenate(
        [jnp.broadcast_to(p['g_card_embed'][None], (B, 1, 128)), tokens,
         jnp.zeros((B, 7, 128), jnp.float32)], axis=1)
    wlist = []
    for i in range(2):
        for nme in ['wq', 'bq', 'wk', 'bk', 'wv', 'bv', 'wo', 'bo']:
            w = p[f'tl{i}_{nme}']
            wlist.append(w.reshape(1, 128) if w.ndim == 1 else w)
        for nme in ['ln1_s', 'ln1_b', 'ln2_s', 'ln2_b']:
            wlist.append(p[f'tl{i}_{nme}'].reshape(1, 128))
        wlist.append(p[f'tl{i}_mw1'])
        wlist.append(p[f'tl{i}_mw2'])
    wlist.append(p['enc_ln_s'].reshape(1, 128))
    wlist.append(p['enc_ln_b'].reshape(1, 128))
    f_cards = _c2_call(x0, wlist)

    ae16_offs, _ = _offsets(_AE16)
    ae8_offs, _ = _offsets(_AE8)
    a16 = _clip_off(ae[:, :, 0:5], _AE16, ae16_offs)
    a8 = _clip_off(ae[:, :, 5:11], _AE8, ae8_offs)
    bd16 = _blockdiag(p, _AE16, 16, 56)
    bd8 = _blockdiag(p, _AE8, 8, 80)
    f_h = _h_call(histg, a16, a8, bd16, bd8)

    ge4tabs = [('ge_turn', 20), ('ge_phase', 11), ('ge_first', 2),
               ('ge_myturn', 2)]
    ge4_offs, _ = _offsets(ge4tabs)
    g2o = _clip_off(g2, ge4tabs, ge4_offs)
    ge4bd = _blockdiag(p, ge4tabs, 16, 40)
    hnd = jnp.concatenate([g3[:, 1:2], g3[:, 8:9]], axis=1)
    cntw = jnp.pad(p['ge_count'], ((0, 4), (0, 0)))
    hndw = jnp.pad(p['ge_hand'], ((0, 4), (0, 0)))
    g1f = global_[:, :4].astype(jnp.float32)
    fg = _d_call(g1f, g2o, g3, hnd, pts, ivs, p['ge_num_w'], p['ge_lp_w'],
                 p['ge_olp_w'], ge4bd, cntw, hndw,
                 p['ge_ln_s'].reshape(1, 256), p['ge_ln_b'].reshape(1, 256),
                 p['g_mlp_w1'], p['g_mlp_w2'], p['g_fc_w'],
                 p['g_ln_s'].reshape(1, 128), p['g_ln_b'].reshape(1, 128))

    return (f_cards, fg, f_h)


# speedup vs baseline: 1.3166x; 1.0207x over previous
import functools

import jax
import jax.numpy as jnp
from jax import lax
from jax.experimental import pallas as pl
from jax.experimental.pallas import tpu as pltpu
from jax.experimental.pallas import tpu_sc as plsc

_pallas_call = pl.pallas_call

_AE16 = [('ae_msg', 30), ('ae_act', 13), ('ae_yesno', 3), ('ae_phase', 4),
         ('ae_cancel', 3)]
_AE8 = [('ae_finish', 3), ('ae_pos', 9), ('ae_opt', 6), ('ae_num', 13),
        ('ae_place', 31), ('ae_attrib', 10)]
_FEAT = [('ce_owner', 2), ('ce_pos', 9), ('ce_over', 2), ('ce_attr', 8),
         ('ce_race', 27), ('ce_level', 14), ('ce_counter', 16), ('ce_neg', 3)]

_EPS = 1e-6


def _lnorm(x, s, b):
    m = jnp.mean(x, -1, keepdims=True)
    d = x - m
    v = jnp.mean(d * d, -1, keepdims=True)
    return d * lax.rsqrt(v + _EPS) * s + b


def _bin_consts():
    p1 = jnp.linspace(0, 8000, 25, dtype=jnp.float32)[1:]
    p2 = jnp.linspace(8000, 32000, 9, dtype=jnp.float32)[1:]
    points = jnp.concatenate([p1, p2])
    intervals = jnp.concatenate([points[0:1], points[1:] - points[:-1]])
    return points.reshape(1, 32), intervals.reshape(1, 32)


def _offsets(tabs):
    offs, t = [], 0
    for _, r in tabs:
        offs.append(t)
        t += r
    return offs, t


def _blockdiag(params, tabs, w, rows_pad):
    offs, total = _offsets(tabs)
    m = jnp.zeros((rows_pad, w * len(tabs)), jnp.float32)
    for j, (n, r) in enumerate(tabs):
        m = m.at[offs[j]:offs[j] + r, j * w:(j + 1) * w].set(params[n])
    return m


def _clip_off(idx, tabs, offs):
    his = jnp.array([r - 1 for _, r in tabs], jnp.int32)
    return jnp.clip(idx, 0, his) + jnp.array(offs, jnp.int32)


def _tables_body(emb, w1, w2, s, b, hw1, hw2, tcard, thist):
    e = emb[...]
    a = jnp.maximum(jnp.dot(e, w1[...], preferred_element_type=jnp.float32), 0.)
    tcard[...] = _lnorm(jnp.dot(a, w2[...], preferred_element_type=jnp.float32),
                        s[...], b[...])
    ah = jnp.maximum(jnp.dot(e, hw1[...], preferred_element_type=jnp.float32), 0.)
    thist[...] = jnp.dot(ah, hw2[...], preferred_element_type=jnp.float32)


def _tables_call(emb, w1, w2, s, b, hw1, hw2):
    R = 128
    n = emb.shape[0] // R
    full = lambda shp: pl.BlockSpec(shp, lambda i: (0, 0))
    return _pallas_call(
        _tables_body,
        grid=(n,),
        in_specs=[pl.BlockSpec((R, 1024), lambda i: (i, 0)),
                  full((1024, 128)), full((128, 32)), full((1, 32)),
                  full((1, 32)), full((1024, 128)), full((128, 128))],
        out_specs=[pl.BlockSpec((R, 32), lambda i: (i, 0)),
                   pl.BlockSpec((R, 128), lambda i: (i, 0))],
        out_shape=[jax.ShapeDtypeStruct((emb.shape[0], 32), jnp.float32),
                   jax.ShapeDtypeStruct((emb.shape[0], 128), jnp.float32)],
    )(emb, w1, w2, s, b, hw1, hw2)


def _c1_body(idg, loci, seqi, fidx, x2, pts, ivs, numw, atkw, defw, typw,
             fs, fb, lstab, fbd, out):
    BB = idg.shape[0]
    T = BB * 80
    x2r = x2[...].reshape(T, 32)
    p = pts[...]
    iv = ivs[...]
    nw = numw[...]

    def num(v):
        bins = jnp.clip((v - p + iv) / iv, 0.0, 1.0)
        return jnp.maximum(jnp.dot(bins, nw, preferred_element_type=jnp.float32), 0.)

    va = x2r[:, 0:1] * 256.0 + x2r[:, 1:2]
    vd = x2r[:, 2:3] * 256.0 + x2r[:, 3:4]
    atk = jnp.dot(num(va), atkw[...], preferred_element_type=jnp.float32)
    dfn = jnp.dot(num(vd), defw[...], preferred_element_type=jnp.float32)
    typ = jnp.dot(x2r[:, 4:32], typw[...], preferred_element_type=jnp.float32)

    li = loci[...].reshape(T, 1)
    si = seqi[...].reshape(T, 1)
    io96 = lax.broadcasted_iota(jnp.int32, (T, 96), 1)
    ohls = ((io96 == li) | (io96 == si + 16)).astype(jnp.float32)
    lsadd = jnp.dot(ohls, lstab[...], preferred_element_type=jnp.float32)

    fr = fidx[...].reshape(T, 8)
    io88 = lax.broadcasted_iota(jnp.int32, (T, 88), 1)
    mh = (io88 == fr[:, 0:1]).astype(jnp.float32)
    for j in range(1, 8):
        mh = mh + (io88 == fr[:, j:j + 1]).astype(jnp.float32)
    feats = jnp.dot(mh, fbd[...], preferred_element_type=jnp.float32)

    xf = jnp.concatenate([feats, atk, dfn, typ], axis=-1)
    xf = _lnorm(xf, fs[...], fb[...])
    tok = jnp.concatenate([idg[...].reshape(T, 32), xf], axis=-1)
    out[...] = (tok + lsadd).reshape(BB, 80, 128)


def _c1_call(idg, loci, seqi, fidx, x2, pts, ivs, numw, atkw, defw, typw,
             fs, fb, lstab, fbd):
    B = idg.shape[0]
    BB = 16
    full = lambda shp: pl.BlockSpec(shp, lambda i: tuple(0 for _ in shp))
    blk3 = lambda d: pl.BlockSpec((BB, 80, d), lambda i: (i, 0, 0))
    return _pallas_call(
        _c1_body,
        grid=(B // BB,),
        in_specs=[blk3(32), blk3(1), blk3(1), blk3(8), blk3(32),
                  full((1, 32)), full((1, 32)), full((32, 16)), full((16, 8)),
                  full((16, 8)), full((28, 16)), full((1, 96)), full((1, 96)),
                  full((96, 128)), full((88, 64))],
        out_specs=blk3(128),
        out_shape=jax.ShapeDtypeStruct((B, 80, 128), jnp.float32),
    )(idg, loci, seqi, fidx, x2, pts, ivs, numw, atkw, defw, typw, fs, fb,
      lstab, fbd)


def _attn(h, wq, bq, wk, bk, wv, bv, wo, bo, BB, T):
    q = jnp.dot(h, wq, preferred_element_type=jnp.float32) + bq
    k = jnp.dot(h, wk, preferred_element_type=jnp.float32) + bk
    v = jnp.dot(h, wv, preferred_element_type=jnp.float32) + bv
    madd = jnp.where(lax.broadcasted_iota(jnp.int32, (1, T), 1) >= 81,
                     -1e30, 0.0)
    outs = []
    for bi in range(BB):
        heads = []
        for hd in range(2):
            qb = q[bi * T:(bi + 1) * T, hd * 64:(hd + 1) * 64]
            kb = k[bi * T:(bi + 1) * T, hd * 64:(hd + 1) * 64]
            vb = v[bi * T:(bi + 1) * T, hd * 64:(hd + 1) * 64]
            a = lax.dot_general(qb, kb, (((1,), (1,)), ((), ())),
                                preferred_element_type=jnp.float32) * 0.125
            e = jnp.exp(a + madd)
            a = e * (1.0 / jnp.sum(e, -1, keepdims=True))
            heads.append(jnp.dot(a, vb, preferred_element_type=jnp.float32))
        outs.append(jnp.concatenate(heads, axis=-1))
    o = jnp.concatenate(outs, axis=0)
    return jnp.dot(o, wo, preferred_element_type=jnp.float32) + bo


def _c2_body(tok, gemb, *refs):
    BB = tok.shape[0]
    T, C = 88, 128
    out = refs[-1]
    w = [r[...] for r in refs[:-1]]
    x = jnp.concatenate(
        [jnp.broadcast_to(gemb[...].reshape(1, 1, C), (BB, 1, C)),
         tok[...], jnp.zeros((BB, 7, C), jnp.float32)],
        axis=1).reshape(BB * T, C)
    for i in range(2):
        (wq, bq, wk, bk, wv, bv, wo, bo, l1s, l1b, l2s, l2b, m1, m2) = \
            w[i * 14:(i + 1) * 14]
        h = _lnorm(x, l1s, l1b)
        x = x + _attn(h, wq, bq, wk, bk, wv, bv, wo, bo, BB, T)
        h = _lnorm(x, l2s, l2b)
        h = jnp.maximum(jnp.dot(h, m1, preferred_element_type=jnp.float32), 0.)
        x = x + jnp.dot(h, m2, preferred_element_type=jnp.float32)
    es, eb = w[28], w[29]
    out[...] = _lnorm(x, es, eb).reshape(BB, T, C)[:, :81, :]


def _c2_call(tokens, gemb, wlist):
    B, C = tokens.shape[0], 128
    BB = 16
    full = lambda shp: pl.BlockSpec(shp, lambda i: tuple(0 for _ in shp))
    return _pallas_call(
        _c2_body,
        grid=(B // BB,),
        in_specs=[pl.BlockSpec((BB, 80, C), lambda i: (i, 0, 0)),
                  full((1, 128))] + [full(w.shape) for w in wlist],
        out_specs=pl.BlockSpec((BB, 81, C), lambda i: (i, 0, 0)),
        out_shape=jax.ShapeDtypeStruct((B, 81, C), jnp.float32),
    )(tokens, gemb, *wlist)


def _h_body(histg, a16, a8, bd16, bd8, out):
    BB, NH = a16.shape[0], a16.shape[1]
    T = BB * NH
    i16 = a16[...].reshape(T, 5)
    i8 = a8[...].reshape(T, 6)
    io56 = lax.broadcasted_iota(jnp.int32, (T, 56), 1)
    mh16 = (io56 == i16[:, 0:1]).astype(jnp.float32)
    for j in range(1, 5):
        mh16 = mh16 + (io56 == i16[:, j:j + 1]).astype(jnp.float32)
    o16 = jnp.dot(mh16, bd16[...], preferred_element_type=jnp.float32)
    io80 = lax.broadcasted_iota(jnp.int32, (T, 80), 1)
    mh8 = (io80 == i8[:, 0:1]).astype(jnp.float32)
    for j in range(1, 6):
        mh8 = mh8 + (io80 == i8[:, j:j + 1]).astype(jnp.float32)
    o8 = jnp.dot(mh8, bd8[...], preferred_element_type=jnp.float32)
    r = jnp.concatenate([histg[...].reshape(T, 128), o16, o8], axis=-1)
    out[...] = r.reshape(BB, NH, 256)


def _h_call(histg, a16, a8, bd16, bd8):
    B, NH = histg.shape[0], histg.shape[1]
    BB = 32
    full = lambda shp: pl.BlockSpec(shp, lambda i: tuple(0 for _ in shp))
    blk3 = lambda d: pl.BlockSpec((BB, NH, d), lambda i: (i, 0, 0))
    return _pallas_call(
        _h_body,
        grid=(B // BB,),
        in_specs=[blk3(128), blk3(5), blk3(6), full((56, 80)), full((80, 48))],
        out_specs=blk3(256),
        out_shape=jax.ShapeDtypeStruct((B, NH, 256), jnp.float32),
    )(histg, a16, a8, bd16, bd8)


def _d_body(g1, g2o, g3c, hnd, pts, ivs, numw, lpw, olpw, ge4bd, cntw, hndw,
            gs, gb, w1, w2, fc, os_, ob_, out):
    B = g1.shape[0]
    p = pts[...]
    iv = ivs[...]
    nw = numw[...]
    g = g1[...]

    def num(v):
        bins = jnp.clip((v - p + iv) / iv, 0.0, 1.0)
        return jnp.maximum(jnp.dot(bins, nw, preferred_element_type=jnp.float32), 0.)

    lp = jnp.dot(num(g[:, 0:1] * 256.0 + g[:, 1:2]), lpw[...],
                 preferred_element_type=jnp.float32)
    olp = jnp.dot(num(g[:, 2:3] * 256.0 + g[:, 3:4]), olpw[...],
                  preferred_element_type=jnp.float32)

    g2 = g2o[...]
    io40 = lax.broadcasted_iota(jnp.int32, (B, 40), 1)
    mh4 = (io40 == g2[:, 0:1]).astype(jnp.float32)
    for j in range(1, 4):
        mh4 = mh4 + (io40 == g2[:, j:j + 1]).astype(jnp.float32)
    ge4 = jnp.dot(mh4, ge4bd[...], preferred_element_type=jnp.float32)

    io104 = lax.broadcasted_iota(jnp.int32, (B, 104), 1)
    cw = cntw[...]
    hw = hndw[...]
    g3 = g3c[...]
    pieces = [lp, olp, ge4]
    for j in range(14):
        oh = (io104 == g3[:, j:j + 1]).astype(jnp.float32)
        pieces.append(jnp.dot(oh, cw, preferred_element_type=jnp.float32))
    hn = hnd[...]
    for j in range(2):
        oh = (io104 == hn[:, j:j + 1]).astype(jnp.float32)
        pieces.append(jnp.dot(oh, hw, preferred_element_type=jnp.float32))
    xg = jnp.concatenate(pieces, axis=-1)
    xg = _lnorm(xg, gs[...], gb[...])
    h = jnp.maximum(jnp.dot(xg, w1[...], preferred_element_type=jnp.float32), 0.)
    fg = xg + jnp.dot(h, w2[...], preferred_element_type=jnp.float32)
    fg = jnp.dot(fg, fc[...], preferred_element_type=jnp.float32)
    out[...] = _lnorm(fg, os_[...], ob_[...])


def _d_call(g1, g2o, g3c, hnd, pts, ivs, numw, lpw, olpw, ge4bd, cntw, hndw,
            gs, gb, w1, w2, fc, os_, ob_):
    B = g1.shape[0]
    full = lambda shp: pl.BlockSpec(shp, lambda: tuple(0 for _ in shp))
    args = (g1, g2o, g3c, hnd, pts, ivs, numw, lpw, olpw, ge4bd, cntw, hndw,
            gs, gb, w1, w2, fc, os_, ob_)
    return _pallas_call(
        _d_body,
        in_specs=[full(a.shape) for a in args],
        out_specs=full((B, 128)),
        out_shape=jax.ShapeDtypeStruct((B, 128), jnp.float32),
    )(*args)


def _pick_chunk(n, d):
    limit = (131072 // (d * 4)) // 8 * 8
    c = min(n, limit)
    c -= c % 8
    while c > 8 and n % c:
        c -= 8
    return c


def _sc_gather2(table_a, idx_a, table_b, idx_b):
    V = 1024
    Da = table_a.shape[1]
    Na = idx_a.shape[0]
    Nb = idx_b.shape[0]
    NS = 16
    npa = Na // NS
    npb = Nb // NS
    CA = 1280
    CB = 256
    HW = 64
    tab_a64 = jnp.pad(table_a, ((0, 0), (0, HW - Da)))
    mesh = plsc.VectorSubcoreMesh(core_axis_name="c", subcore_axis_name="s")

    @functools.partial(
        pl.kernel, mesh=mesh,
        out_type=[jax.ShapeDtypeStruct((Na, Da), jnp.float32),
                  jax.ShapeDtypeStruct((Nb, 2 * HW), jnp.float32)],
        compiler_params=pltpu.CompilerParams(use_tc_tiling_on_sc=False,
                                             needs_layout_passes=False),
        scratch_types=[pltpu.VMEM((V, HW), jnp.float32),
                       pltpu.VMEM((npa,), jnp.int32),
                       pltpu.VMEM((CA, Da), jnp.float32),
                       pltpu.VMEM((CB, HW), jnp.float32)],
    )
    def k(ia_hbm, ta_hbm, ib_hbm, tb_hbm, oa_hbm, ob_hbm,
          tab_v, idx_v, oa_v, ob_v):
        cid = lax.axis_index("c")
        sid = lax.axis_index("s")
        iot = lax.broadcasted_iota(jnp.int32, (16,), 0)

        @pl.when(cid == 0)
        def _card():
            pltpu.sync_copy(ta_hbm, tab_v)
            pltpu.sync_copy(ia_hbm.at[pl.ds(sid * npa, npa)], idx_v)
            for ch in range(npa // CA):
                def body(kk, _):
                    ids = idx_v[pl.ds(ch * CA + kk * 16, 16)]
                    rows = iot + kk * 16
                    for c in range(Da):
                        cv = jnp.full((16,), c, jnp.int32)
                        vals = plsc.load_gather(tab_v, [ids, cv])
                        plsc.store_scatter(oa_v, [rows, cv], vals)
                    return 0
                lax.fori_loop(0, CA // 16, body, 0)
                pltpu.sync_copy(oa_v, oa_hbm.at[pl.ds(sid * npa + ch * CA, CA)])

        @pl.when(cid == 1)
        def _hist():
            pltpu.sync_copy(ib_hbm.at[pl.ds(sid * npb, npb)],
                            idx_v.at[pl.ds(0, npb)])
            for half in range(2):
                pltpu.sync_copy(tb_hbm.at[:, pl.ds(half * HW, HW)], tab_v)
                for ch in range(npb // CB):
                    def body(kk, _):
                        ids = idx_v[pl.ds(ch * CB + kk * 16, 16)]
                        rows = iot + kk * 16
                        for c in range(HW):
                            cv = jnp.full((16,), c, jnp.int32)
                            vals = plsc.load_gather(tab_v, [ids, cv])
                            plsc.store_scatter(ob_v, [rows, cv], vals)
                        return 0
                    lax.fori_loop(0, CB // 16, body, 0)
                    pltpu.sync_copy(
                        ob_v, ob_hbm.at[pl.ds(sid * npb + ch * CB, CB),
                                        pl.ds(half * HW, HW)])

    return k(idx_a, tab_a64, idx_b, table_b)


def kernel(cards_, global_, actions_, h_actions_, params):
    p = params
    B, NCARD = cards_.shape[0], cards_.shape[1]
    NH = h_actions_.shape[1]
    pts, ivs = _bin_consts()

    emb = jnp.pad(p['id_embed'], ((0, 23), (0, 0)))
    tcard, thist = _tables_call(
        emb, p['ce_id_w1'], p['ce_id_w2'],
        p['ce_id_ln_s'].reshape(1, 32), p['ce_id_ln_b'].reshape(1, 32),
        p['h_id_w1'], p['h_id_w2'])

    ids_card = jnp.clip(cards_[:, :, 0] * 256 + cards_[:, :, 1], 0, 1000)
    x1 = cards_[:, :, 2:12]
    h = h_actions_.astype(jnp.int32)
    ids_h = jnp.clip(h[:, :, 0] * 256 + h[:, :, 1], 0, 1000)
    ae = h[:, :, 2:13]
    g2 = global_[:, 4:8].astype(jnp.int32)
    g3 = jnp.clip(global_[:, 8:22].astype(jnp.int32), 0, 99)

    g32, ghist = _sc_gather2(tcard, ids_card.reshape(-1),
                             thist, ids_h.reshape(-1))
    id32g = g32.reshape(B, NCARD, 32)
    histg = ghist.reshape(B, NH, 128)

    loc_i = jnp.clip(x1[:, :, 0:1], 0, 8)
    seq_i = jnp.clip(x1[:, :, 1:2], 0, 75)
    ft_offs, _ = _offsets(_FEAT)
    fidx = _clip_off(x1[:, :, 2:10], _FEAT, ft_offs)

    locp = jnp.pad(_lnorm(p['ce_loc_embed'], p['ce_loc_ln_s'],
                          p['ce_loc_ln_b']), ((0, 7), (0, 0)))
    seqp = jnp.pad(_lnorm(p['ce_seq_embed'], p['ce_seq_ln_s'],
                          p['ce_seq_ln_b']), ((0, 4), (0, 0)))
    lstab = jnp.concatenate([locp, seqp], 0)
    fbd = _blockdiag(p, _FEAT, 8, 88)

    x2 = cards_[:, :, 12:41].astype(jnp.float32)
    x2p = jnp.pad(x2, ((0, 0), (0, 0), (0, 3)))
    typw = jnp.pad(p['ce_type_w'], ((0, 3), (0, 0)))
    tokens = _c1_call(id32g, loc_i, seq_i, fidx, x2p, pts, ivs,
                      p['ce_num_w'], p['ce_atk_w'], p['ce_def_w'], typw,
                      p['ce_f_ln_s'].reshape(1, 96),
                      p['ce_f_ln_b'].reshape(1, 96), lstab, fbd)

    wlist = []
    for i in range(2):
        for nme in ['wq', 'bq', 'wk', 'bk', 'wv', 'bv', 'wo', 'bo']:
            w = p[f'tl{i}_{nme}']
            wlist.append(w.reshape(1, 128) if w.ndim == 1 else w)
        for nme in ['ln1_s', 'ln1_b', 'ln2_s', 'ln2_b']:
            wlist.append(p[f'tl{i}_{nme}'].reshape(1, 128))
        wlist.append(p[f'tl{i}_mw1'])
        wlist.append(p[f'tl{i}_mw2'])
    wlist.append(p['enc_ln_s'].reshape(1, 128))
    wlist.append(p['enc_ln_b'].reshape(1, 128))
    f_cards = _c2_call(tokens, p['g_card_embed'], wlist)

    ae16_offs, _ = _offsets(_AE16)
    ae8_offs, _ = _offsets(_AE8)
    a16 = _clip_off(ae[:, :, 0:5], _AE16, ae16_offs)
    a8 = _clip_off(ae[:, :, 5:11], _AE8, ae8_offs)
    bd16 = _blockdiag(p, _AE16, 16, 56)
    bd8 = _blockdiag(p, _AE8, 8, 80)
    f_h = _h_call(histg, a16, a8, bd16, bd8)

    ge4tabs = [('ge_turn', 20), ('ge_phase', 11), ('ge_first', 2),
               ('ge_myturn', 2)]
    ge4_offs, _ = _offsets(ge4tabs)
    g2o = _clip_off(g2, ge4tabs, ge4_offs)
    ge4bd = _blockdiag(p, ge4tabs, 16, 40)
    hnd = jnp.concatenate([g3[:, 1:2], g3[:, 8:9]], axis=1)
    cntw = jnp.pad(p['ge_count'], ((0, 4), (0, 0)))
    hndw = jnp.pad(p['ge_hand'], ((0, 4), (0, 0)))
    g1f = global_[:, :4].astype(jnp.float32)
    fg = _d_call(g1f, g2o, g3, hnd, pts, ivs, p['ge_num_w'], p['ge_lp_w'],
                 p['ge_olp_w'], ge4bd, cntw, hndw,
                 p['ge_ln_s'].reshape(1, 256), p['ge_ln_b'].reshape(1, 256),
                 p['g_mlp_w1'], p['g_mlp_w2'], p['g_fc_w'],
                 p['g_ln_s'].reshape(1, 128), p['g_ln_b'].reshape(1, 128))

    return (f_cards, fg, f_h)
